# baseline (speedup 1.0000x reference)
import functools

import jax
import jax.numpy as jnp
from jax import lax
from jax.experimental import pallas as pl
from jax.experimental.pallas import tpu as pltpu
from jax.experimental.pallas import tpu_sc as plsc

N = 10000
NE = 5000
NNZ = 320000
D = 128
DEG_E = 0.5
DEG_V = 0.0625

NC = 2
NS = 16
NW = NC * NS
K = 125
PH = 5
PCH = 16
NE_PAD = 8192
ROWS_B = NE_PAD // NW
ZROWS_A = NE_PAD // NS
N_PAD = 10240
ZROWS_C = N_PAD // NS
CROWS = NE_PAD // D

_mesh = plsc.VectorSubcoreMesh(
    core_axis_name="c", subcore_axis_name="s", num_cores=NC, num_subcores=NS)
_params = pltpu.CompilerParams(needs_layout_passes=False)


@functools.partial(
    pl.kernel,
    out_type=(
        jax.ShapeDtypeStruct((NC, NE_PAD, D), jnp.float32),
        jax.ShapeDtypeStruct((NC, CROWS, D), jnp.float32),
    ),
    mesh=_mesh,
    scratch_types=[
        pltpu.VMEM_SHARED((NE_PAD, D), jnp.float32),
        pltpu.VMEM_SHARED((CROWS, D), jnp.float32),
        pltpu.VMEM((PCH, K), jnp.int32),
        pltpu.VMEM((PCH, K), jnp.int32),
        pltpu.VMEM((K, D), jnp.float32),
        pltpu.VMEM((K, D), jnp.float32),
        pltpu.VMEM((CROWS, D), jnp.float32),
        pltpu.VMEM((CROWS,), jnp.int32),
        pltpu.SemaphoreType.DMA,
        pltpu.SemaphoreType.DMA,
        pltpu.SemaphoreType.DMA,
        pltpu.SemaphoreType.DMA,
    ],
    compiler_params=_params,
)
def _edge_accum(x_hbm, v_hbm, e_hbm, z_hbm, out_hbm, cnt_hbm,
                acc, acc_cnt, vidx, eidx, rows_a, rows_b, cnt2, idx_id,
                sem_ga, sem_gb, sem_sa, sem_sb):
    c = lax.axis_index("c")
    s = lax.axis_index("s")
    wid = c * NS + s
    pltpu.sync_copy(z_hbm.at[pl.ds(s * ZROWS_A, ZROWS_A)],
                    acc.at[pl.ds(s * ZROWS_A, ZROWS_A)])
    pltpu.sync_copy(z_hbm.at[pl.ds(0, CROWS)], cnt2)

    @pl.when(s == 0)
    def _():
        pltpu.sync_copy(z_hbm.at[pl.ds(0, CROWS)], acc_cnt)

    def init_idx(g, carry):
        idx_id[pl.ds(g * 16, 16)] = lax.iota(jnp.int32, 16) + g * 16
        return carry

    lax.fori_loop(0, CROWS // 16, init_idx, 0)
    plsc.subcore_barrier()

    ones16 = jnp.ones((16,), jnp.float32)
    tail_mask = lax.iota(jnp.int32, 16) >= (16 - K % 16)

    def count(g):
        for j in range(K // 16):
            ids = eidx[g, pl.ds(j * 16, 16)]
            plsc.addupdate_scatter(cnt2, [ids >> 7, ids & 127], ones16)
        ids = eidx[g, pl.ds(K - 16, 16)]
        plsc.addupdate_scatter(cnt2, [ids >> 7, ids & 127], ones16, mask=tail_mask)

    def phase(p, carry):
        pltpu.sync_copy(v_hbm.at[wid, p], vidx)
        pltpu.sync_copy(e_hbm.at[wid, p], eidx)
        pltpu.async_copy(x_hbm.at[vidx.at[0]], rows_a, sem_ga)

        def pair(t, carry2):
            ga = 2 * t
            gb = 2 * t + 1
            pltpu.make_async_copy(x_hbm.at[vidx.at[ga]], rows_a, sem_ga).wait()

            @pl.when(t > 0)
            def _():
                pltpu.make_async_copy(rows_b, acc.at[eidx.at[ga]], sem_sb).wait()

            pltpu.async_copy(x_hbm.at[vidx.at[gb]], rows_b, sem_gb)
            pltpu.async_copy(rows_a, acc.at[eidx.at[ga]], sem_sa, add=True)
            count(ga)
            pltpu.make_async_copy(x_hbm.at[vidx.at[gb]], rows_b, sem_gb).wait()
            pltpu.make_async_copy(rows_a, acc.at[eidx.at[ga]], sem_sa).wait()
            nxt = jnp.minimum(gb + 1, PCH - 1)

            @pl.when(gb + 1 < PCH)
            def _():
                pltpu.async_copy(x_hbm.at[vidx.at[nxt]], rows_a, sem_ga)

            pltpu.async_copy(rows_b, acc.at[eidx.at[gb]], sem_sb, add=True)
            count(gb)
            return carry2

        lax.fori_loop(0, PCH // 2, pair, 0)
        pltpu.make_async_copy(rows_b, acc.at[eidx.at[PCH - 1]], sem_sb).wait()
        return carry

    lax.fori_loop(0, PH, phase, 0)
    pltpu.sync_copy(cnt2, acc_cnt.at[idx_id], add=True)
    plsc.subcore_barrier()
    pltpu.sync_copy(acc.at[pl.ds(s * ZROWS_A, ZROWS_A)],
                    out_hbm.at[c, pl.ds(s * ZROWS_A, ZROWS_A)])

    @pl.when(s == 0)
    def _():
        pltpu.sync_copy(acc_cnt, cnt_hbm.at[c])


@functools.partial(
    pl.kernel,
    out_type=jax.ShapeDtypeStruct((NE_PAD, D), jnp.float32),
    mesh=_mesh,
    scratch_types=[
        pltpu.VMEM((ROWS_B, D), jnp.float32),
        pltpu.VMEM((ROWS_B, D), jnp.float32),
        pltpu.VMEM((CROWS, D), jnp.float32),
        pltpu.VMEM((CROWS, D), jnp.float32),
        pltpu.VMEM((ROWS_B, D), jnp.float32),
    ],
    compiler_params=_params,
)
def _edge_combine(parts_hbm, cnt_hbm, out_hbm, p0, p1, c0, c1, o):
    c = lax.axis_index("c")
    s = lax.axis_index("s")
    base = (c * NS + s) * ROWS_B
    pltpu.sync_copy(parts_hbm.at[0, pl.ds(base, ROWS_B)], p0)
    pltpu.sync_copy(parts_hbm.at[1, pl.ds(base, ROWS_B)], p1)
    pltpu.sync_copy(cnt_hbm.at[0], c0)
    pltpu.sync_copy(cnt_hbm.at[1], c1)

    def grp(t, carry):
        fp = base + t * 16
        row = fp >> 7
        col = fp & 127
        cnt16 = c0[row, pl.ds(col, 16)] + c1[row, pl.ds(col, 16)]
        sv16 = DEG_E / jnp.maximum(cnt16, 1.0)
        for j in range(16):
            scale = jnp.full((16,), sv16[j], dtype=jnp.float32)
            r = t * 16 + j
            for cc in range(D // 16):
                slc = pl.ds(cc * 16, 16)
                o[r, slc] = (p0[r, slc] + p1[r, slc]) * scale
        return carry

    lax.fori_loop(0, ROWS_B // 16, grp, 0)
    pltpu.sync_copy(o, out_hbm.at[pl.ds(base, ROWS_B)])


@functools.partial(
    pl.kernel,
    out_type=jax.ShapeDtypeStruct((NC, N_PAD, D), jnp.float32),
    mesh=_mesh,
    scratch_types=[
        pltpu.VMEM_SHARED((N_PAD, D), jnp.float32),
        pltpu.VMEM((PCH, K), jnp.int32),
        pltpu.VMEM((PCH, K), jnp.int32),
        pltpu.VMEM((K, D), jnp.float32),
        pltpu.VMEM((K, D), jnp.float32),
        pltpu.SemaphoreType.DMA,
        pltpu.SemaphoreType.DMA,
        pltpu.SemaphoreType.DMA,
        pltpu.SemaphoreType.DMA,
    ],
    compiler_params=_params,
)
def _node_accum(xe_hbm, v_hbm, e_hbm, z_hbm, out_hbm, acc, vidx, eidx,
                rows_a, rows_b, sem_ga, sem_gb, sem_sa, sem_sb):
    c = lax.axis_index("c")
    s = lax.axis_index("s")
    wid = c * NS + s
    pltpu.sync_copy(z_hbm.at[pl.ds(s * ZROWS_C, ZROWS_C)],
                    acc.at[pl.ds(s * ZROWS_C, ZROWS_C)])
    plsc.subcore_barrier()

    def phase(p, carry):
        pltpu.sync_copy(v_hbm.at[wid, p], vidx)
        pltpu.sync_copy(e_hbm.at[wid, p], eidx)
        pltpu.async_copy(xe_hbm.at[eidx.at[0]], rows_a, sem_ga)

        def pair(t, carry2):
            ga = 2 * t
            gb = 2 * t + 1
            pltpu.make_async_copy(xe_hbm.at[eidx.at[ga]], rows_a, sem_ga).wait()

            @pl.when(t > 0)
            def _():
                pltpu.make_async_copy(rows_b, acc.at[vidx.at[ga]], sem_sb).wait()

            pltpu.async_copy(xe_hbm.at[eidx.at[gb]], rows_b, sem_gb)
            pltpu.async_copy(rows_a, acc.at[vidx.at[ga]], sem_sa, add=True)
            pltpu.make_async_copy(xe_hbm.at[eidx.at[gb]], rows_b, sem_gb).wait()
            pltpu.make_async_copy(rows_a, acc.at[vidx.at[ga]], sem_sa).wait()
            nxt = jnp.minimum(gb + 1, PCH - 1)

            @pl.when(gb + 1 < PCH)
            def _():
                pltpu.async_copy(xe_hbm.at[eidx.at[nxt]], rows_a, sem_ga)

            pltpu.async_copy(rows_b, acc.at[vidx.at[gb]], sem_sb, add=True)
            return carry2

        lax.fori_loop(0, PCH // 2, pair, 0)
        pltpu.make_async_copy(rows_b, acc.at[vidx.at[PCH - 1]], sem_sb).wait()
        return carry

    lax.fori_loop(0, PH, phase, 0)
    plsc.subcore_barrier()
    pltpu.sync_copy(acc.at[pl.ds(s * ZROWS_C, ZROWS_C)],
                    out_hbm.at[c, pl.ds(s * ZROWS_C, ZROWS_C)])


BLK = 1000


def _dense_body(scal_ref, parts_ref, x0_ref, w_ref, out_ref):
    a = scal_ref[0]
    b = scal_ref[1]
    xv = (parts_ref[0] + parts_ref[1]) * (2.0 * DEG_V)
    xi = (1.0 - a) * xv + a * x0_ref[...]
    out_ref[...] = (1.0 - b) * xi + b * lax.dot_general(
        xi, w_ref[...], (((1,), (1,)), ((), ())),
        preferred_element_type=jnp.float32)


def _dense(parts, x0, w, alpha, beta):
    scal = jnp.stack([alpha, beta]).astype(jnp.float32)
    return pl.pallas_call(
        _dense_body,
        grid=(N // BLK,),
        in_specs=[
            pl.BlockSpec(memory_space=pltpu.SMEM),
            pl.BlockSpec((NC, BLK, D), lambda i: (0, i, 0)),
            pl.BlockSpec((BLK, D), lambda i: (i, 0)),
            pl.BlockSpec((D, D), lambda i: (0, 0)),
        ],
        out_specs=pl.BlockSpec((BLK, D), lambda i: (i, 0)),
        out_shape=jax.ShapeDtypeStruct((N, D), jnp.float32),
        compiler_params=pltpu.CompilerParams(
            dimension_semantics=("arbitrary",)),
    )(scal, parts, x0, w)


def kernel(X, vertex, edges, X0, W, alpha, beta):
    v2 = vertex.reshape(NW, PH, PCH, K)
    e2 = edges.reshape(NW, PH, PCH, K)
    z_a = jnp.zeros((NE_PAD, D), jnp.float32)
    z_c = jnp.zeros((N_PAD, D), jnp.float32)
    se_parts, cnts = _edge_accum(X, v2, e2, z_a)
    xe = _edge_combine(se_parts, cnts)
    xv_parts = _node_accum(xe, v2, e2, z_c)
    return _dense(xv_parts, X0, W, alpha, beta)


# speedup vs baseline: 1.0190x; 1.0190x over previous
import functools

import jax
import jax.numpy as jnp
from jax import lax
from jax.experimental import pallas as pl
from jax.experimental.pallas import tpu as pltpu
from jax.experimental.pallas import tpu_sc as plsc

N = 10000
NE = 5000
NNZ = 320000
D = 128
DEG_E = 0.5
DEG_V = 0.0625

NC = 2
NS = 16
NW = NC * NS
K = 125
PH = 5
PCH = 16
NE_PAD = 8192
ROWS_B = NE_PAD // NW
ZROWS_A = NE_PAD // NS
N_PAD = 10240
ZROWS_C = N_PAD // NS
CROWS = NE_PAD // D

_mesh = plsc.VectorSubcoreMesh(
    core_axis_name="c", subcore_axis_name="s", num_cores=NC, num_subcores=NS)
_params = pltpu.CompilerParams(needs_layout_passes=False)


@functools.partial(
    pl.kernel,
    out_type=(
        jax.ShapeDtypeStruct((NC, NE_PAD, D), jnp.float32),
        jax.ShapeDtypeStruct((NC, CROWS, D), jnp.float32),
    ),
    mesh=_mesh,
    scratch_types=[
        pltpu.VMEM_SHARED((NE_PAD, D), jnp.float32),
        pltpu.VMEM_SHARED((CROWS, D), jnp.float32),
        pltpu.VMEM((PCH, K), jnp.int32),
        pltpu.VMEM((PCH, K), jnp.int32),
        pltpu.VMEM((K, D), jnp.float32),
        pltpu.VMEM((K, D), jnp.float32),
        pltpu.VMEM((CROWS, D), jnp.float32),
        pltpu.VMEM((CROWS,), jnp.int32),
        pltpu.SemaphoreType.DMA,
        pltpu.SemaphoreType.DMA,
        pltpu.SemaphoreType.DMA,
        pltpu.SemaphoreType.DMA,
    ],
    compiler_params=_params,
)
def _edge_accum(x_hbm, v_hbm, e_hbm, z_hbm, out_hbm, cnt_hbm,
                acc, acc_cnt, vidx, eidx, rows_a, rows_b, cnt2, idx_id,
                sem_ga, sem_gb, sem_sa, sem_sb):
    c = lax.axis_index("c")
    s = lax.axis_index("s")
    wid = c * NS + s
    pltpu.sync_copy(v_hbm.at[wid, 0], vidx)
    pltpu.sync_copy(e_hbm.at[wid, 0], eidx)
    pltpu.async_copy(x_hbm.at[vidx.at[0]], rows_a, sem_ga)
    pltpu.sync_copy(z_hbm.at[pl.ds(s * ZROWS_A, ZROWS_A)],
                    acc.at[pl.ds(s * ZROWS_A, ZROWS_A)])
    pltpu.sync_copy(z_hbm.at[pl.ds(0, CROWS)], cnt2)

    @pl.when(s == 0)
    def _():
        pltpu.sync_copy(z_hbm.at[pl.ds(0, CROWS)], acc_cnt)

    def init_idx(g, carry):
        idx_id[pl.ds(g * 16, 16)] = lax.iota(jnp.int32, 16) + g * 16
        return carry

    lax.fori_loop(0, CROWS // 16, init_idx, 0)
    plsc.subcore_barrier()

    ones16 = jnp.ones((16,), jnp.float32)
    tail_mask = lax.iota(jnp.int32, 16) >= (16 - K % 16)

    def count(g):
        for j in range(K // 16):
            ids = eidx[g, pl.ds(j * 16, 16)]
            plsc.addupdate_scatter(cnt2, [ids >> 7, ids & 127], ones16)
        ids = eidx[g, pl.ds(K - 16, 16)]
        plsc.addupdate_scatter(cnt2, [ids >> 7, ids & 127], ones16, mask=tail_mask)

    def phase(p, carry):
        @pl.when(p > 0)
        def _():
            pltpu.sync_copy(v_hbm.at[wid, p], vidx)
            pltpu.sync_copy(e_hbm.at[wid, p], eidx)
            pltpu.async_copy(x_hbm.at[vidx.at[0]], rows_a, sem_ga)

        def pair(t, carry2):
            ga = 2 * t
            gb = 2 * t + 1
            pltpu.make_async_copy(x_hbm.at[vidx.at[ga]], rows_a, sem_ga).wait()

            @pl.when(t > 0)
            def _():
                pltpu.make_async_copy(rows_b, acc.at[eidx.at[ga]], sem_sb).wait()

            pltpu.async_copy(x_hbm.at[vidx.at[gb]], rows_b, sem_gb)
            pltpu.async_copy(rows_a, acc.at[eidx.at[ga]], sem_sa, add=True)
            count(ga)
            pltpu.make_async_copy(x_hbm.at[vidx.at[gb]], rows_b, sem_gb).wait()
            pltpu.make_async_copy(rows_a, acc.at[eidx.at[ga]], sem_sa).wait()
            nxt = jnp.minimum(gb + 1, PCH - 1)

            @pl.when(gb + 1 < PCH)
            def _():
                pltpu.async_copy(x_hbm.at[vidx.at[nxt]], rows_a, sem_ga)

            pltpu.async_copy(rows_b, acc.at[eidx.at[gb]], sem_sb, add=True)
            count(gb)
            return carry2

        lax.fori_loop(0, PCH // 2, pair, 0)
        pltpu.make_async_copy(rows_b, acc.at[eidx.at[PCH - 1]], sem_sb).wait()
        return carry

    lax.fori_loop(0, PH, phase, 0)
    pltpu.sync_copy(cnt2, acc_cnt.at[idx_id], add=True)
    plsc.subcore_barrier()
    pltpu.sync_copy(acc.at[pl.ds(s * ZROWS_A, ZROWS_A)],
                    out_hbm.at[c, pl.ds(s * ZROWS_A, ZROWS_A)])

    @pl.when(s == 0)
    def _():
        pltpu.sync_copy(acc_cnt, cnt_hbm.at[c])


@functools.partial(
    pl.kernel,
    out_type=jax.ShapeDtypeStruct((NE_PAD, D), jnp.float32),
    mesh=_mesh,
    scratch_types=[
        pltpu.VMEM((ROWS_B, D), jnp.float32),
        pltpu.VMEM((ROWS_B, D), jnp.float32),
        pltpu.VMEM((CROWS, D), jnp.float32),
        pltpu.VMEM((CROWS, D), jnp.float32),
        pltpu.VMEM((ROWS_B, D), jnp.float32),
        pltpu.SemaphoreType.DMA,
        pltpu.SemaphoreType.DMA,
        pltpu.SemaphoreType.DMA,
        pltpu.SemaphoreType.DMA,
    ],
    compiler_params=_params,
)
def _edge_combine(parts_hbm, cnt_hbm, out_hbm, p0, p1, c0, c1, o,
                  m0, m1, m2, m3):
    c = lax.axis_index("c")
    s = lax.axis_index("s")
    base = (c * NS + s) * ROWS_B
    pltpu.async_copy(parts_hbm.at[0, pl.ds(base, ROWS_B)], p0, m0)
    pltpu.async_copy(parts_hbm.at[1, pl.ds(base, ROWS_B)], p1, m1)
    pltpu.async_copy(cnt_hbm.at[0], c0, m2)
    pltpu.async_copy(cnt_hbm.at[1], c1, m3)
    pltpu.make_async_copy(parts_hbm.at[0, pl.ds(base, ROWS_B)], p0, m0).wait()
    pltpu.make_async_copy(parts_hbm.at[1, pl.ds(base, ROWS_B)], p1, m1).wait()
    pltpu.make_async_copy(cnt_hbm.at[0], c0, m2).wait()
    pltpu.make_async_copy(cnt_hbm.at[1], c1, m3).wait()

    def grp(t, carry):
        fp = base + t * 16
        row = fp >> 7
        col = fp & 127
        cnt16 = c0[row, pl.ds(col, 16)] + c1[row, pl.ds(col, 16)]
        sv16 = DEG_E / jnp.maximum(cnt16, 1.0)
        for j in range(16):
            scale = jnp.full((16,), sv16[j], dtype=jnp.float32)
            r = t * 16 + j
            for cc in range(D // 16):
                slc = pl.ds(cc * 16, 16)
                o[r, slc] = (p0[r, slc] + p1[r, slc]) * scale
        return carry

    lax.fori_loop(0, ROWS_B // 16, grp, 0)
    pltpu.sync_copy(o, out_hbm.at[pl.ds(base, ROWS_B)])


@functools.partial(
    pl.kernel,
    out_type=jax.ShapeDtypeStruct((NC, N_PAD, D), jnp.float32),
    mesh=_mesh,
    scratch_types=[
        pltpu.VMEM_SHARED((N_PAD, D), jnp.float32),
        pltpu.VMEM((PCH, K), jnp.int32),
        pltpu.VMEM((PCH, K), jnp.int32),
        pltpu.VMEM((K, D), jnp.float32),
        pltpu.VMEM((K, D), jnp.float32),
        pltpu.SemaphoreType.DMA,
        pltpu.SemaphoreType.DMA,
        pltpu.SemaphoreType.DMA,
        pltpu.SemaphoreType.DMA,
    ],
    compiler_params=_params,
)
def _node_accum(xe_hbm, v_hbm, e_hbm, z_hbm, out_hbm, acc, vidx, eidx,
                rows_a, rows_b, sem_ga, sem_gb, sem_sa, sem_sb):
    c = lax.axis_index("c")
    s = lax.axis_index("s")
    wid = c * NS + s
    pltpu.sync_copy(v_hbm.at[wid, 0], vidx)
    pltpu.sync_copy(e_hbm.at[wid, 0], eidx)
    pltpu.async_copy(xe_hbm.at[eidx.at[0]], rows_a, sem_ga)
    pltpu.sync_copy(z_hbm.at[pl.ds(s * ZROWS_C, ZROWS_C)],
                    acc.at[pl.ds(s * ZROWS_C, ZROWS_C)])
    plsc.subcore_barrier()

    def phase(p, carry):
        @pl.when(p > 0)
        def _():
            pltpu.sync_copy(v_hbm.at[wid, p], vidx)
            pltpu.sync_copy(e_hbm.at[wid, p], eidx)
            pltpu.async_copy(xe_hbm.at[eidx.at[0]], rows_a, sem_ga)

        def pair(t, carry2):
            ga = 2 * t
            gb = 2 * t + 1
            pltpu.make_async_copy(xe_hbm.at[eidx.at[ga]], rows_a, sem_ga).wait()

            @pl.when(t > 0)
            def _():
                pltpu.make_async_copy(rows_b, acc.at[vidx.at[ga]], sem_sb).wait()

            pltpu.async_copy(xe_hbm.at[eidx.at[gb]], rows_b, sem_gb)
            pltpu.async_copy(rows_a, acc.at[vidx.at[ga]], sem_sa, add=True)
            pltpu.make_async_copy(xe_hbm.at[eidx.at[gb]], rows_b, sem_gb).wait()
            pltpu.make_async_copy(rows_a, acc.at[vidx.at[ga]], sem_sa).wait()
            nxt = jnp.minimum(gb + 1, PCH - 1)

            @pl.when(gb + 1 < PCH)
            def _():
                pltpu.async_copy(xe_hbm.at[eidx.at[nxt]], rows_a, sem_ga)

            pltpu.async_copy(rows_b, acc.at[vidx.at[gb]], sem_sb, add=True)
            return carry2

        lax.fori_loop(0, PCH // 2, pair, 0)
        pltpu.make_async_copy(rows_b, acc.at[vidx.at[PCH - 1]], sem_sb).wait()
        return carry

    lax.fori_loop(0, PH, phase, 0)
    plsc.subcore_barrier()
    pltpu.sync_copy(acc.at[pl.ds(s * ZROWS_C, ZROWS_C)],
                    out_hbm.at[c, pl.ds(s * ZROWS_C, ZROWS_C)])


BLK = 1000


def _dense_body(scal_ref, parts_ref, x0_ref, w_ref, out_ref):
    a = scal_ref[0]
    b = scal_ref[1]
    xv = (parts_ref[0] + parts_ref[1]) * (2.0 * DEG_V)
    xi = (1.0 - a) * xv + a * x0_ref[...]
    out_ref[...] = (1.0 - b) * xi + b * lax.dot_general(
        xi, w_ref[...], (((1,), (1,)), ((), ())),
        preferred_element_type=jnp.float32)


def _dense(parts, x0, w, alpha, beta):
    scal = jnp.stack([alpha, beta]).astype(jnp.float32)
    return pl.pallas_call(
        _dense_body,
        grid=(N // BLK,),
        in_specs=[
            pl.BlockSpec(memory_space=pltpu.SMEM),
            pl.BlockSpec((NC, BLK, D), lambda i: (0, i, 0)),
            pl.BlockSpec((BLK, D), lambda i: (i, 0)),
            pl.BlockSpec((D, D), lambda i: (0, 0)),
        ],
        out_specs=pl.BlockSpec((BLK, D), lambda i: (i, 0)),
        out_shape=jax.ShapeDtypeStruct((N, D), jnp.float32),
        compiler_params=pltpu.CompilerParams(
            dimension_semantics=("arbitrary",)),
    )(scal, parts, x0, w)


def kernel(X, vertex, edges, X0, W, alpha, beta):
    v2 = vertex.reshape(NW, PH, PCH, K)
    e2 = edges.reshape(NW, PH, PCH, K)
    z_a = jnp.zeros((NE_PAD, D), jnp.float32)
    z_c = jnp.zeros((N_PAD, D), jnp.float32)
    se_parts, cnts = _edge_accum(X, v2, e2, z_a)
    xe = _edge_combine(se_parts, cnts)
    xv_parts = _node_accum(xe, v2, e2, z_c)
    return _dense(xv_parts, X0, W, alpha, beta)


# speedup vs baseline: 1.0244x; 1.0053x over previous
import functools

import jax
import jax.numpy as jnp
from jax import lax
from jax.experimental import pallas as pl
from jax.experimental.pallas import tpu as pltpu
from jax.experimental.pallas import tpu_sc as plsc

N = 10000
NE = 5000
NNZ = 320000
D = 128
DEG_E = 0.5
DEG_V = 0.0625

NC = 2
NS = 16
NW = NC * NS
K = 125
PH = 5
PCH = 16
PHA = 2
PCHA = 40
NE_PAD = 8192
ROWS_B = NE_PAD // NW
ZROWS_A = NE_PAD // NS
N_PAD = 10240
ZROWS_C = N_PAD // NS
CROWS = NE_PAD // D

_mesh = plsc.VectorSubcoreMesh(
    core_axis_name="c", subcore_axis_name="s", num_cores=NC, num_subcores=NS)
_params = pltpu.CompilerParams(needs_layout_passes=False)


@functools.partial(
    pl.kernel,
    out_type=(
        jax.ShapeDtypeStruct((NC, NE_PAD, D), jnp.float32),
        jax.ShapeDtypeStruct((NC, CROWS, D), jnp.float32),
    ),
    mesh=_mesh,
    scratch_types=[
        pltpu.VMEM_SHARED((NE_PAD, D), jnp.float32),
        pltpu.VMEM_SHARED((CROWS, D), jnp.float32),
        pltpu.VMEM((PCHA, K), jnp.int32),
        pltpu.VMEM((PCHA, K), jnp.int32),
        pltpu.VMEM((K, D), jnp.float32),
        pltpu.VMEM((K, D), jnp.float32),
        pltpu.VMEM((CROWS, D), jnp.float32),
        pltpu.VMEM((CROWS,), jnp.int32),
        pltpu.SemaphoreType.DMA,
        pltpu.SemaphoreType.DMA,
        pltpu.SemaphoreType.DMA,
        pltpu.SemaphoreType.DMA,
    ],
    compiler_params=_params,
)
def _edge_accum(x_hbm, v_hbm, e_hbm, z_hbm, out_hbm, cnt_hbm,
                acc, acc_cnt, vidx, eidx, rows_a, rows_b, cnt2, idx_id,
                sem_ga, sem_gb, sem_sa, sem_sb):
    c = lax.axis_index("c")
    s = lax.axis_index("s")
    wid = c * NS + s
    pltpu.sync_copy(v_hbm.at[wid, 0], vidx)
    pltpu.sync_copy(e_hbm.at[wid, 0], eidx)
    pltpu.async_copy(x_hbm.at[vidx.at[0]], rows_a, sem_ga)
    pltpu.sync_copy(z_hbm.at[pl.ds(s * ZROWS_A, ZROWS_A)],
                    acc.at[pl.ds(s * ZROWS_A, ZROWS_A)])
    pltpu.sync_copy(z_hbm.at[pl.ds(0, CROWS)], cnt2)

    @pl.when(s == 0)
    def _():
        pltpu.sync_copy(z_hbm.at[pl.ds(0, CROWS)], acc_cnt)

    def init_idx(g, carry):
        idx_id[pl.ds(g * 16, 16)] = lax.iota(jnp.int32, 16) + g * 16
        return carry

    lax.fori_loop(0, CROWS // 16, init_idx, 0)
    plsc.subcore_barrier()

    ones16 = jnp.ones((16,), jnp.float32)
    tail_mask = lax.iota(jnp.int32, 16) >= (16 - K % 16)

    def count(g):
        for j in range(K // 16):
            ids = eidx[g, pl.ds(j * 16, 16)]
            plsc.addupdate_scatter(cnt2, [ids >> 7, ids & 127], ones16)
        ids = eidx[g, pl.ds(K - 16, 16)]
        plsc.addupdate_scatter(cnt2, [ids >> 7, ids & 127], ones16, mask=tail_mask)

    def phase(p, carry):
        @pl.when(p > 0)
        def _():
            pltpu.sync_copy(v_hbm.at[wid, p], vidx)
            pltpu.sync_copy(e_hbm.at[wid, p], eidx)
            pltpu.async_copy(x_hbm.at[vidx.at[0]], rows_a, sem_ga)

        def pair(t, carry2):
            ga = 2 * t
            gb = 2 * t + 1
            pltpu.make_async_copy(x_hbm.at[vidx.at[ga]], rows_a, sem_ga).wait()

            @pl.when(t > 0)
            def _():
                pltpu.make_async_copy(rows_b, acc.at[eidx.at[ga]], sem_sb).wait()

            pltpu.async_copy(x_hbm.at[vidx.at[gb]], rows_b, sem_gb)
            pltpu.async_copy(rows_a, acc.at[eidx.at[ga]], sem_sa, add=True)
            count(ga)
            pltpu.make_async_copy(x_hbm.at[vidx.at[gb]], rows_b, sem_gb).wait()
            pltpu.make_async_copy(rows_a, acc.at[eidx.at[ga]], sem_sa).wait()
            nxt = jnp.minimum(gb + 1, PCHA - 1)

            @pl.when(gb + 1 < PCHA)
            def _():
                pltpu.async_copy(x_hbm.at[vidx.at[nxt]], rows_a, sem_ga)

            pltpu.async_copy(rows_b, acc.at[eidx.at[gb]], sem_sb, add=True)
            count(gb)
            return carry2

        lax.fori_loop(0, PCHA // 2, pair, 0)
        pltpu.make_async_copy(rows_b, acc.at[eidx.at[PCHA - 1]], sem_sb).wait()
        return carry

    lax.fori_loop(0, PHA, phase, 0)
    pltpu.sync_copy(cnt2, acc_cnt.at[idx_id], add=True)
    plsc.subcore_barrier()
    pltpu.sync_copy(acc.at[pl.ds(s * ZROWS_A, ZROWS_A)],
                    out_hbm.at[c, pl.ds(s * ZROWS_A, ZROWS_A)])

    @pl.when(s == 0)
    def _():
        pltpu.sync_copy(acc_cnt, cnt_hbm.at[c])


@functools.partial(
    pl.kernel,
    out_type=jax.ShapeDtypeStruct((NE_PAD, D), jnp.float32),
    mesh=_mesh,
    scratch_types=[
        pltpu.VMEM((ROWS_B, D), jnp.float32),
        pltpu.VMEM((ROWS_B, D), jnp.float32),
        pltpu.VMEM((CROWS, D), jnp.float32),
        pltpu.VMEM((CROWS, D), jnp.float32),
        pltpu.VMEM((ROWS_B, D), jnp.float32),
        pltpu.SemaphoreType.DMA,
        pltpu.SemaphoreType.DMA,
        pltpu.SemaphoreType.DMA,
        pltpu.SemaphoreType.DMA,
    ],
    compiler_params=_params,
)
def _edge_combine(parts_hbm, cnt_hbm, out_hbm, p0, p1, c0, c1, o,
                  m0, m1, m2, m3):
    c = lax.axis_index("c")
    s = lax.axis_index("s")
    base = (c * NS + s) * ROWS_B
    pltpu.async_copy(parts_hbm.at[0, pl.ds(base, ROWS_B)], p0, m0)
    pltpu.async_copy(parts_hbm.at[1, pl.ds(base, ROWS_B)], p1, m1)
    pltpu.async_copy(cnt_hbm.at[0], c0, m2)
    pltpu.async_copy(cnt_hbm.at[1], c1, m3)
    pltpu.make_async_copy(parts_hbm.at[0, pl.ds(base, ROWS_B)], p0, m0).wait()
    pltpu.make_async_copy(parts_hbm.at[1, pl.ds(base, ROWS_B)], p1, m1).wait()
    pltpu.make_async_copy(cnt_hbm.at[0], c0, m2).wait()
    pltpu.make_async_copy(cnt_hbm.at[1], c1, m3).wait()

    def grp(t, carry):
        fp = base + t * 16
        row = fp >> 7
        col = fp & 127
        cnt16 = c0[row, pl.ds(col, 16)] + c1[row, pl.ds(col, 16)]
        sv16 = DEG_E / jnp.maximum(cnt16, 1.0)
        for j in range(16):
            scale = jnp.full((16,), sv16[j], dtype=jnp.float32)
            r = t * 16 + j
            for cc in range(D // 16):
                slc = pl.ds(cc * 16, 16)
                o[r, slc] = (p0[r, slc] + p1[r, slc]) * scale
        return carry

    lax.fori_loop(0, ROWS_B // 16, grp, 0)
    pltpu.sync_copy(o, out_hbm.at[pl.ds(base, ROWS_B)])


@functools.partial(
    pl.kernel,
    out_type=jax.ShapeDtypeStruct((NC, N_PAD, D), jnp.float32),
    mesh=_mesh,
    scratch_types=[
        pltpu.VMEM_SHARED((N_PAD, D), jnp.float32),
        pltpu.VMEM((PCH, K), jnp.int32),
        pltpu.VMEM((PCH, K), jnp.int32),
        pltpu.VMEM((K, D), jnp.float32),
        pltpu.VMEM((K, D), jnp.float32),
        pltpu.SemaphoreType.DMA,
        pltpu.SemaphoreType.DMA,
        pltpu.SemaphoreType.DMA,
        pltpu.SemaphoreType.DMA,
    ],
    compiler_params=_params,
)
def _node_accum(xe_hbm, v_hbm, e_hbm, z_hbm, out_hbm, acc, vidx, eidx,
                rows_a, rows_b, sem_ga, sem_gb, sem_sa, sem_sb):
    c = lax.axis_index("c")
    s = lax.axis_index("s")
    wid = c * NS + s
    pltpu.sync_copy(v_hbm.at[wid, 0], vidx)
    pltpu.sync_copy(e_hbm.at[wid, 0], eidx)
    pltpu.async_copy(xe_hbm.at[eidx.at[0]], rows_a, sem_ga)
    pltpu.sync_copy(z_hbm.at[pl.ds(s * ZROWS_C, ZROWS_C)],
                    acc.at[pl.ds(s * ZROWS_C, ZROWS_C)])
    plsc.subcore_barrier()

    def phase(p, carry):
        @pl.when(p > 0)
        def _():
            pltpu.sync_copy(v_hbm.at[wid, p], vidx)
            pltpu.sync_copy(e_hbm.at[wid, p], eidx)
            pltpu.async_copy(xe_hbm.at[eidx.at[0]], rows_a, sem_ga)

        def pair(t, carry2):
            ga = 2 * t
            gb = 2 * t + 1
            pltpu.make_async_copy(xe_hbm.at[eidx.at[ga]], rows_a, sem_ga).wait()

            @pl.when(t > 0)
            def _():
                pltpu.make_async_copy(rows_b, acc.at[vidx.at[ga]], sem_sb).wait()

            pltpu.async_copy(xe_hbm.at[eidx.at[gb]], rows_b, sem_gb)
            pltpu.async_copy(rows_a, acc.at[vidx.at[ga]], sem_sa, add=True)
            pltpu.make_async_copy(xe_hbm.at[eidx.at[gb]], rows_b, sem_gb).wait()
            pltpu.make_async_copy(rows_a, acc.at[vidx.at[ga]], sem_sa).wait()
            nxt = jnp.minimum(gb + 1, PCH - 1)

            @pl.when(gb + 1 < PCH)
            def _():
                pltpu.async_copy(xe_hbm.at[eidx.at[nxt]], rows_a, sem_ga)

            pltpu.async_copy(rows_b, acc.at[vidx.at[gb]], sem_sb, add=True)
            return carry2

        lax.fori_loop(0, PCH // 2, pair, 0)
        pltpu.make_async_copy(rows_b, acc.at[vidx.at[PCH - 1]], sem_sb).wait()
        return carry

    lax.fori_loop(0, PH, phase, 0)
    plsc.subcore_barrier()
    pltpu.sync_copy(acc.at[pl.ds(s * ZROWS_C, ZROWS_C)],
                    out_hbm.at[c, pl.ds(s * ZROWS_C, ZROWS_C)])


BLK = 1000


def _dense_body(scal_ref, parts_ref, x0_ref, w_ref, out_ref):
    a = scal_ref[0]
    b = scal_ref[1]
    xv = (parts_ref[0] + parts_ref[1]) * (2.0 * DEG_V)
    xi = (1.0 - a) * xv + a * x0_ref[...]
    out_ref[...] = (1.0 - b) * xi + b * lax.dot_general(
        xi, w_ref[...], (((1,), (1,)), ((), ())),
        preferred_element_type=jnp.float32)


def _dense(parts, x0, w, alpha, beta):
    scal = jnp.stack([alpha, beta]).astype(jnp.float32)
    return pl.pallas_call(
        _dense_body,
        grid=(N // BLK,),
        in_specs=[
            pl.BlockSpec(memory_space=pltpu.SMEM),
            pl.BlockSpec((NC, BLK, D), lambda i: (0, i, 0)),
            pl.BlockSpec((BLK, D), lambda i: (i, 0)),
            pl.BlockSpec((D, D), lambda i: (0, 0)),
        ],
        out_specs=pl.BlockSpec((BLK, D), lambda i: (i, 0)),
        out_shape=jax.ShapeDtypeStruct((N, D), jnp.float32),
        compiler_params=pltpu.CompilerParams(
            dimension_semantics=("arbitrary",)),
    )(scal, parts, x0, w)


def kernel(X, vertex, edges, X0, W, alpha, beta):
    v2a = vertex.reshape(NW, PHA, PCHA, K)
    e2a = edges.reshape(NW, PHA, PCHA, K)
    v2 = vertex.reshape(NW, PH, PCH, K)
    e2 = edges.reshape(NW, PH, PCH, K)
    z_a = jnp.zeros((NE_PAD, D), jnp.float32)
    z_c = jnp.zeros((N_PAD, D), jnp.float32)
    se_parts, cnts = _edge_accum(X, v2a, e2a, z_a)
    xe = _edge_combine(se_parts, cnts)
    xv_parts = _node_accum(xe, v2, e2, z_c)
    return _dense(xv_parts, X0, W, alpha, beta)
